# baseline (speedup 1.0000x reference)
import jax
import jax.numpy as jnp
from jax.experimental import pallas as pl
from jax.experimental.pallas import tpu as pltpu

_HIDDEN = 64
_N = 128
_ITERS = 2
_BB = 8
_LOG2E = 1.4426950408889634


def _dot(a, b):
    return jax.lax.dot_general(a, b, (((1,), (0,)), ((), ())),
                               preferred_element_type=jnp.float32)


def _bf(v):
    return v.astype(jnp.bfloat16)


def _mpnn_kernel(x_ref, Wemb_ref, bemb_ref, Wadj_ref, Wmsg_ref, bmsg_ref,
                 Wz_ref, Uz_ref, bz_ref, Wr_ref, Ur_ref, br_ref,
                 Wh_ref, Uh_ref, bh_ref, Wro_ref, bro_ref,
                 out_ref, A_ref):
    H = _HIDDEN
    h = jnp.tanh(_dot(x_ref[...], Wemb_ref[...]) + bemb_ref[...])
    ones16 = jnp.ones((_N, H), jnp.bfloat16)
    for t in range(_ITERS):
        h16 = _bf(h)
        hw16 = _bf(_dot(h16, Wadj_ref[t]))
        pre16 = _bf(_dot(h16, Wmsg_ref[t]) + bmsg_ref[t])
        msgs = []
        for b in range(_BB):
            sl = slice(b * _N, (b + 1) * _N)
            logits = jax.lax.dot_general(
                hw16[sl, :], h16[sl, :], (((1,), (1,)), ((), ())),
                preferred_element_type=jnp.float32)
            e = jnp.exp2(logits)
            pre_aug = jnp.concatenate([pre16[sl, :], ones16], axis=1)
            s = _dot(_bf(e), pre_aug)
            inv = pl.reciprocal(s[:, H:], approx=True)
            msgs.append(s[:, :H] * inv)
            if t == _ITERS - 1:
                A_ref[b] = e * jnp.concatenate([inv, inv], axis=1)
        msg = jnp.tanh(jnp.concatenate(msgs, axis=0))
        msg16 = _bf(msg)
        z = jax.nn.sigmoid(_dot(msg16, Wz_ref[t]) + _dot(h16, Uz_ref[t]) + bz_ref[t])
        r = jax.nn.sigmoid(_dot(msg16, Wr_ref[t]) + _dot(h16, Ur_ref[t]) + br_ref[t])
        htil = jnp.tanh(_dot(msg16, Wh_ref[t]) + _dot(_bf(r * h), Uh_ref[t]) + bh_ref[t])
        h = (1.0 - z) * h + z * htil
    rows = jax.lax.broadcasted_iota(jnp.int32, (_BB, _BB * _N), 0)
    cols = jax.lax.broadcasted_iota(jnp.int32, (_BB, _BB * _N), 1)
    poolmask = jnp.where(cols // _N == rows, jnp.float32(1), jnp.float32(0))
    pooled = _dot(poolmask, h)
    out_ref[...] = jnp.tanh(_dot(_bf(pooled), Wro_ref[...]) + bro_ref[...])


def kernel(jets, W_emb, b_emb, W_adj, W_msg, b_msg,
           Wz, Uz, bz, Wr, Ur, br, Wh, Uh, bh, W_ro, b_ro):
    B, N, F = jets.shape
    H = _HIDDEN
    x = jnp.concatenate([jets, jnp.ones((B, N, 1), jets.dtype)], axis=-1)
    x = x.reshape(B * N, F + 1).astype(jnp.bfloat16)
    W_adj_s = W_adj * jnp.float32(_LOG2E / (float(H) ** 0.5))
    bf = lambda v: v.astype(jnp.bfloat16)

    def rep(ix):
        return pl.BlockSpec(ix, lambda i: (0,) * len(ix))

    out, A = pl.pallas_call(
        _mpnn_kernel,
        grid=(B // _BB,),
        in_specs=[
            pl.BlockSpec((_BB * N, F + 1), lambda i: (i, 0)),
            rep((F + 1, H)),
            rep((1, H)),
            rep((_ITERS, H, H)),
            rep((_ITERS, H, H)),
            rep((_ITERS, 1, H)),
            rep((_ITERS, H, H)), rep((_ITERS, H, H)), rep((_ITERS, 1, H)),
            rep((_ITERS, H, H)), rep((_ITERS, H, H)), rep((_ITERS, 1, H)),
            rep((_ITERS, H, H)), rep((_ITERS, H, H)), rep((_ITERS, 1, H)),
            rep((H, H)),
            rep((1, H)),
        ],
        out_specs=[
            pl.BlockSpec((_BB, H), lambda i: (i, 0)),
            pl.BlockSpec((_BB, N, N), lambda i: (i, 0, 0)),
        ],
        out_shape=[
            jax.ShapeDtypeStruct((B, H), jnp.float32),
            jax.ShapeDtypeStruct((B, N, N), jnp.float32),
        ],
        compiler_params=pltpu.CompilerParams(
            dimension_semantics=("parallel",)),
    )(x, bf(W_emb), b_emb.reshape(1, H),
      bf(W_adj_s), bf(W_msg), b_msg.reshape(_ITERS, 1, H),
      bf(Wz), bf(Uz), bz.reshape(_ITERS, 1, H),
      bf(Wr), bf(Ur), br.reshape(_ITERS, 1, H),
      bf(Wh), bf(Uh), bh.reshape(_ITERS, 1, H),
      bf(W_ro), b_ro.reshape(1, H))
    return (out, A)


# speedup vs baseline: 1.0805x; 1.0805x over previous
import jax
import jax.numpy as jnp
from jax.experimental import pallas as pl
from jax.experimental.pallas import tpu as pltpu

_HIDDEN = 64
_N = 128
_ITERS = 2
_BB = 8
_LOG2E = 1.4426950408889634


def _dot(a, b):
    return jax.lax.dot_general(a, b, (((1,), (0,)), ((), ())),
                               preferred_element_type=jnp.float32)


def _mpnn_kernel(x_ref, Wemb_ref, bemb_ref, Wadj_ref, Wmsg_ref, bmsg_ref,
                 Wz_ref, Uz_ref, bz_ref, Wr_ref, Ur_ref, br_ref,
                 Wh_ref, Uh_ref, bh_ref, Wro_ref, bro_ref,
                 out_ref, A_ref):
    H = _HIDDEN
    h = jnp.tanh(_dot(x_ref[...], Wemb_ref[...]) + bemb_ref[...])
    ones_blk = jnp.ones((_N, H), jnp.float32)
    for t in range(_ITERS):
        hw = _dot(h, Wadj_ref[t])
        pre = _dot(h, Wmsg_ref[t]) + bmsg_ref[t]
        msgs = []
        for b in range(_BB):
            sl = slice(b * _N, (b + 1) * _N)
            logits = jax.lax.dot_general(
                hw[sl, :], h[sl, :], (((1,), (1,)), ((), ())),
                preferred_element_type=jnp.float32)
            e = jnp.exp2(logits)
            pre_aug = jnp.concatenate([pre[sl, :], ones_blk], axis=1)
            s = _dot(e, pre_aug)
            inv = pl.reciprocal(s[:, H:], approx=True)
            msgs.append(s[:, :H] * inv)
            if t == _ITERS - 1:
                A_ref[b] = e * jnp.concatenate([inv, inv], axis=1)
        msg = jnp.tanh(jnp.concatenate(msgs, axis=0))
        z = jax.nn.sigmoid(_dot(msg, Wz_ref[t]) + _dot(h, Uz_ref[t]) + bz_ref[t])
        r = jax.nn.sigmoid(_dot(msg, Wr_ref[t]) + _dot(h, Ur_ref[t]) + br_ref[t])
        htil = jnp.tanh(_dot(msg, Wh_ref[t]) + _dot(r * h, Uh_ref[t]) + bh_ref[t])
        h = (1.0 - z) * h + z * htil
    rows = jax.lax.broadcasted_iota(jnp.int32, (_BB, _BB * _N), 0)
    cols = jax.lax.broadcasted_iota(jnp.int32, (_BB, _BB * _N), 1)
    poolmask = jnp.where(cols // _N == rows, jnp.float32(1), jnp.float32(0))
    pooled = _dot(poolmask, h)
    out_ref[...] = jnp.tanh(_dot(pooled, Wro_ref[...]) + bro_ref[...])


def kernel(jets, W_emb, b_emb, W_adj, W_msg, b_msg,
           Wz, Uz, bz, Wr, Ur, br, Wh, Uh, bh, W_ro, b_ro):
    B, N, F = jets.shape
    H = _HIDDEN
    x = jnp.concatenate([jets, jnp.ones((B, N, 1), jets.dtype)], axis=-1)
    x = x.reshape(B * N, F + 1)
    W_adj_s = W_adj * jnp.float32(_LOG2E / (float(H) ** 0.5))

    def rep(ix):
        return pl.BlockSpec(ix, lambda i: (0,) * len(ix))

    out, A = pl.pallas_call(
        _mpnn_kernel,
        grid=(B // _BB,),
        in_specs=[
            pl.BlockSpec((_BB * N, F + 1), lambda i: (i, 0)),
            rep((F + 1, H)),
            rep((1, H)),
            rep((_ITERS, H, H)),
            rep((_ITERS, H, H)),
            rep((_ITERS, 1, H)),
            rep((_ITERS, H, H)), rep((_ITERS, H, H)), rep((_ITERS, 1, H)),
            rep((_ITERS, H, H)), rep((_ITERS, H, H)), rep((_ITERS, 1, H)),
            rep((_ITERS, H, H)), rep((_ITERS, H, H)), rep((_ITERS, 1, H)),
            rep((H, H)),
            rep((1, H)),
        ],
        out_specs=[
            pl.BlockSpec((_BB, H), lambda i: (i, 0)),
            pl.BlockSpec((_BB, N, N), lambda i: (i, 0, 0)),
        ],
        out_shape=[
            jax.ShapeDtypeStruct((B, H), jnp.float32),
            jax.ShapeDtypeStruct((B, N, N), jnp.float32),
        ],
        compiler_params=pltpu.CompilerParams(
            dimension_semantics=("parallel",)),
    )(x, W_emb, b_emb.reshape(1, H),
      W_adj_s, W_msg, b_msg.reshape(_ITERS, 1, H),
      Wz, Uz, bz.reshape(_ITERS, 1, H),
      Wr, Ur, br.reshape(_ITERS, 1, H),
      Wh, Uh, bh.reshape(_ITERS, 1, H),
      W_ro, b_ro.reshape(1, H))
    return (out, A)


# speedup vs baseline: 1.5171x; 1.4040x over previous
import jax
import jax.numpy as jnp
from jax.experimental import pallas as pl
from jax.experimental.pallas import tpu as pltpu

_HIDDEN = 64
_N = 128
_ITERS = 2
_BB = 8
_LOG2E = 1.4426950408889634


def _dot(a, b):
    return jax.lax.dot_general(a, b, (((1,), (0,)), ((), ())),
                               preferred_element_type=jnp.float32)


def _mpnn_kernel(x_ref, Wemb_ref, bemb_ref, Wadj_ref, Wmsg_ref, bmsg_ref,
                 Wz_ref, Uz_ref, bz_ref, Wr_ref, Ur_ref, br_ref,
                 Wh_ref, Uh_ref, bh_ref, Wro_ref, bro_ref,
                 out_ref, A_ref):
    H = _HIDDEN
    h = jnp.tanh(_dot(x_ref[...], Wemb_ref[...]) + bemb_ref[...])
    ones_blk = jnp.ones((_N, H), jnp.float32)
    for t in range(_ITERS):
        hw = _dot(h, Wadj_ref[t])
        pre = _dot(h, Wmsg_ref[t]) + bmsg_ref[t]
        es = []
        for b in range(_BB):
            sl = slice(b * _N, (b + 1) * _N)
            logits = jax.lax.dot_general(
                hw[sl, :], h[sl, :], (((1,), (1,)), ((), ())),
                preferred_element_type=jnp.float32)
            es.append(jnp.exp2(logits))
        ss = []
        for b in range(_BB):
            sl = slice(b * _N, (b + 1) * _N)
            pre_aug = jnp.concatenate([pre[sl, :], ones_blk], axis=1)
            ss.append(_dot(es[b], pre_aug))
        msgs = []
        for b in range(_BB):
            inv = 1.0 / ss[b][:, H:]
            msgs.append(ss[b][:, :H] * inv)
            if t == _ITERS - 1:
                A_ref[b] = es[b] * jnp.concatenate([inv, inv], axis=1)
        msg = jnp.tanh(jnp.concatenate(msgs, axis=0))
        z = jax.nn.sigmoid(_dot(msg, Wz_ref[t]) + _dot(h, Uz_ref[t]) + bz_ref[t])
        r = jax.nn.sigmoid(_dot(msg, Wr_ref[t]) + _dot(h, Ur_ref[t]) + br_ref[t])
        htil = jnp.tanh(_dot(msg, Wh_ref[t]) + _dot(r * h, Uh_ref[t]) + bh_ref[t])
        h = (1.0 - z) * h + z * htil
    pooled = jnp.concatenate(
        [jnp.sum(h[b * _N:(b + 1) * _N, :], axis=0, keepdims=True)
         for b in range(_BB)], axis=0)
    out_ref[...] = jnp.tanh(_dot(pooled, Wro_ref[...]) + bro_ref[...])


def kernel(jets, W_emb, b_emb, W_adj, W_msg, b_msg,
           Wz, Uz, bz, Wr, Ur, br, Wh, Uh, bh, W_ro, b_ro):
    B, N, F = jets.shape
    H = _HIDDEN
    x = jnp.concatenate([jets, jnp.ones((B, N, 1), jets.dtype)], axis=-1)
    x = x.reshape(B * N, F + 1)
    W_adj_s = W_adj * jnp.float32(_LOG2E / (float(H) ** 0.5))

    def rep(ix):
        return pl.BlockSpec(ix, lambda i: (0,) * len(ix))

    out, A = pl.pallas_call(
        _mpnn_kernel,
        grid=(B // _BB,),
        in_specs=[
            pl.BlockSpec((_BB * N, F + 1), lambda i: (i, 0)),
            rep((F + 1, H)),
            rep((1, H)),
            rep((_ITERS, H, H)),
            rep((_ITERS, H, H)),
            rep((_ITERS, 1, H)),
            rep((_ITERS, H, H)), rep((_ITERS, H, H)), rep((_ITERS, 1, H)),
            rep((_ITERS, H, H)), rep((_ITERS, H, H)), rep((_ITERS, 1, H)),
            rep((_ITERS, H, H)), rep((_ITERS, H, H)), rep((_ITERS, 1, H)),
            rep((H, H)),
            rep((1, H)),
        ],
        out_specs=[
            pl.BlockSpec((_BB, H), lambda i: (i, 0)),
            pl.BlockSpec((_BB, N, N), lambda i: (i, 0, 0)),
        ],
        out_shape=[
            jax.ShapeDtypeStruct((B, H), jnp.float32),
            jax.ShapeDtypeStruct((B, N, N), jnp.float32),
        ],
        compiler_params=pltpu.CompilerParams(
            dimension_semantics=("parallel",)),
    )(x, W_emb, b_emb.reshape(1, H),
      W_adj_s, W_msg, b_msg.reshape(_ITERS, 1, H),
      Wz, Uz, bz.reshape(_ITERS, 1, H),
      Wr, Ur, br.reshape(_ITERS, 1, H),
      Wh, Uh, bh.reshape(_ITERS, 1, H),
      W_ro, b_ro.reshape(1, H))
    return (out, A)


# speedup vs baseline: 1.7340x; 1.1430x over previous
import jax
import jax.numpy as jnp
from jax.experimental import pallas as pl
from jax.experimental.pallas import tpu as pltpu

_HIDDEN = 64
_N = 128
_ITERS = 2
_BB = 8
_LOG2E = 1.4426950408889634


def _dot(a, b):
    return jax.lax.dot_general(a, b, (((1,), (0,)), ((), ())),
                               preferred_element_type=jnp.float32)


def _bf(v):
    return v.astype(jnp.bfloat16)


def _mpnn_kernel(x_ref, Wemb_ref, bemb_ref, Wap_ref, bmsg_ref,
                 Wzr_ref, bzr_ref, Wcand_ref, bh_ref, Wro_ref, bro_ref,
                 out_ref, A_ref):
    H = _HIDDEN
    h = jnp.tanh(_dot(x_ref[...], Wemb_ref[...]) + bemb_ref[...])
    ones_blk = jnp.ones((_N, H), jnp.bfloat16)
    for t in range(_ITERS):
        h16 = _bf(h)
        hp = _dot(h16, Wap_ref[t])
        hw16 = _bf(hp[:, :H])
        pre16 = _bf(hp[:, H:] + bmsg_ref[t])
        es = []
        for b in range(_BB):
            sl = slice(b * _N, (b + 1) * _N)
            logits = jax.lax.dot_general(
                hw16[sl, :], h16[sl, :], (((1,), (1,)), ((), ())),
                preferred_element_type=jnp.float32)
            es.append(jnp.exp2(logits))
        ss = []
        for b in range(_BB):
            sl = slice(b * _N, (b + 1) * _N)
            pre_aug = jnp.concatenate([pre16[sl, :], ones_blk], axis=1)
            ss.append(_dot(_bf(es[b]), pre_aug))
        msgs = []
        for b in range(_BB):
            inv = 1.0 / ss[b][:, H:]
            msgs.append(ss[b][:, :H] * inv)
            if t == _ITERS - 1:
                A_ref[b] = es[b] * jnp.concatenate([inv, inv], axis=1)
        msg = jnp.tanh(jnp.concatenate(msgs, axis=0))
        msg16 = _bf(msg)
        mh16 = jnp.concatenate([msg16, h16], axis=1)
        zr = jnp.tanh(_dot(mh16, Wzr_ref[t]) + bzr_ref[t])
        z = 0.5 + 0.5 * zr[:, :H]
        r = 0.5 + 0.5 * zr[:, H:]
        mrh16 = jnp.concatenate([msg16, _bf(r * h)], axis=1)
        htil = jnp.tanh(_dot(mrh16, Wcand_ref[t]) + bh_ref[t])
        h = h + z * (htil - h)
    pooled = jnp.concatenate(
        [jnp.sum(h[b * _N:(b + 1) * _N, :], axis=0, keepdims=True)
         for b in range(_BB)], axis=0)
    out_ref[...] = jnp.tanh(_dot(pooled, Wro_ref[...]) + bro_ref[...])


def kernel(jets, W_emb, b_emb, W_adj, W_msg, b_msg,
           Wz, Uz, bz, Wr, Ur, br, Wh, Uh, bh, W_ro, b_ro):
    B, N, F = jets.shape
    H = _HIDDEN
    x = jnp.concatenate([jets, jnp.ones((B, N, 1), jets.dtype)], axis=-1)
    x = x.reshape(B * N, F + 1)
    W_adj_s = W_adj * jnp.float32(_LOG2E / (float(H) ** 0.5))
    W_ap = jnp.concatenate([W_adj_s, W_msg], axis=2)
    W_zr = 0.5 * jnp.concatenate(
        [jnp.concatenate([Wz, Wr], axis=2),
         jnp.concatenate([Uz, Ur], axis=2)], axis=1)
    b_zr = 0.5 * jnp.concatenate([bz, br], axis=1)
    W_cand = jnp.concatenate([Wh, Uh], axis=1)
    bf = lambda v: v.astype(jnp.bfloat16)

    def rep(ix):
        return pl.BlockSpec(ix, lambda i: (0,) * len(ix))

    out, A = pl.pallas_call(
        _mpnn_kernel,
        grid=(B // _BB,),
        in_specs=[
            pl.BlockSpec((_BB * N, F + 1), lambda i: (i, 0)),
            rep((F + 1, H)),
            rep((1, H)),
            rep((_ITERS, H, 2 * H)),
            rep((_ITERS, 1, H)),
            rep((_ITERS, 2 * H, 2 * H)),
            rep((_ITERS, 1, 2 * H)),
            rep((_ITERS, 2 * H, H)),
            rep((_ITERS, 1, H)),
            rep((H, H)),
            rep((1, H)),
        ],
        out_specs=[
            pl.BlockSpec((_BB, H), lambda i: (i, 0)),
            pl.BlockSpec((_BB, N, N), lambda i: (i, 0, 0)),
        ],
        out_shape=[
            jax.ShapeDtypeStruct((B, H), jnp.float32),
            jax.ShapeDtypeStruct((B, N, N), jnp.float32),
        ],
        compiler_params=pltpu.CompilerParams(
            dimension_semantics=("parallel",)),
    )(bf(x), bf(W_emb), b_emb.reshape(1, H),
      bf(W_ap), b_msg.reshape(_ITERS, 1, H),
      bf(W_zr), b_zr.reshape(_ITERS, 1, 2 * H),
      bf(W_cand), bh.reshape(_ITERS, 1, H),
      bf(W_ro), b_ro.reshape(1, H))
    return (out, A)


# speedup vs baseline: 2.0971x; 1.2094x over previous
import jax
import jax.numpy as jnp
from jax.experimental import pallas as pl
from jax.experimental.pallas import tpu as pltpu

_HIDDEN = 64
_N = 128
_ITERS = 2
_BB = 16
_LOG2E = 1.4426950408889634


def _dot(a, b):
    return jax.lax.dot_general(a, b, (((1,), (0,)), ((), ())),
                               preferred_element_type=jnp.float32)


def _bf(v):
    return v.astype(jnp.bfloat16)


def _mpnn_kernel(x_ref, Wemb_ref, bemb_ref, Wap_ref, bmsg_ref,
                 Wzr_ref, bzr_ref, Wcand_ref, bh_ref, Wro_ref, bro_ref,
                 out_ref, A_ref):
    H = _HIDDEN
    h = jnp.tanh(_dot(x_ref[...], Wemb_ref[...]) + bemb_ref[...])
    ones_blk = jnp.ones((_N, H), jnp.bfloat16)
    for t in range(_ITERS):
        h16 = _bf(h)
        hp = _dot(h16, Wap_ref[t])
        hw16 = _bf(hp[:, :H])
        pre16 = _bf(hp[:, H:] + bmsg_ref[t])
        es = []
        for b in range(_BB):
            sl = slice(b * _N, (b + 1) * _N)
            logits = jax.lax.dot_general(
                hw16[sl, :], h16[sl, :], (((1,), (1,)), ((), ())),
                preferred_element_type=jnp.float32)
            es.append(jnp.exp2(logits))
        ss = []
        for b in range(_BB):
            sl = slice(b * _N, (b + 1) * _N)
            pre_aug = jnp.concatenate([pre16[sl, :], ones_blk], axis=1)
            ss.append(_dot(_bf(es[b]), pre_aug))
        msgs = []
        for b in range(_BB):
            inv = 1.0 / ss[b][:, H:]
            msgs.append(ss[b][:, :H] * inv)
            if t == _ITERS - 1:
                A_ref[b] = es[b] * jnp.concatenate([inv, inv], axis=1)
        msg = jnp.tanh(jnp.concatenate(msgs, axis=0))
        msg16 = _bf(msg)
        mh16 = jnp.concatenate([msg16, h16], axis=1)
        zr = jnp.tanh(_dot(mh16, Wzr_ref[t]) + bzr_ref[t])
        z = 0.5 + 0.5 * zr[:, :H]
        r = 0.5 + 0.5 * zr[:, H:]
        mrh16 = jnp.concatenate([msg16, _bf(r * h)], axis=1)
        htil = jnp.tanh(_dot(mrh16, Wcand_ref[t]) + bh_ref[t])
        h = h + z * (htil - h)
    pooled = jnp.concatenate(
        [jnp.sum(h[b * _N:(b + 1) * _N, :], axis=0, keepdims=True)
         for b in range(_BB)], axis=0)
    out_ref[...] = jnp.tanh(_dot(pooled, Wro_ref[...]) + bro_ref[...])


def kernel(jets, W_emb, b_emb, W_adj, W_msg, b_msg,
           Wz, Uz, bz, Wr, Ur, br, Wh, Uh, bh, W_ro, b_ro):
    B, N, F = jets.shape
    H = _HIDDEN
    x = jnp.concatenate([jets, jnp.ones((B, N, 1), jets.dtype)], axis=-1)
    x = x.reshape(B * N, F + 1)
    W_adj_s = W_adj * jnp.float32(_LOG2E / (float(H) ** 0.5))
    W_ap = jnp.concatenate([W_adj_s, W_msg], axis=2)
    W_zr = 0.5 * jnp.concatenate(
        [jnp.concatenate([Wz, Wr], axis=2),
         jnp.concatenate([Uz, Ur], axis=2)], axis=1)
    b_zr = 0.5 * jnp.concatenate([bz, br], axis=1)
    W_cand = jnp.concatenate([Wh, Uh], axis=1)
    bf = lambda v: v.astype(jnp.bfloat16)

    def rep(ix):
        return pl.BlockSpec(ix, lambda i: (0,) * len(ix))

    out, A = pl.pallas_call(
        _mpnn_kernel,
        grid=(B // _BB,),
        in_specs=[
            pl.BlockSpec((_BB * N, F + 1), lambda i: (i, 0)),
            rep((F + 1, H)),
            rep((1, H)),
            rep((_ITERS, H, 2 * H)),
            rep((_ITERS, 1, H)),
            rep((_ITERS, 2 * H, 2 * H)),
            rep((_ITERS, 1, 2 * H)),
            rep((_ITERS, 2 * H, H)),
            rep((_ITERS, 1, H)),
            rep((H, H)),
            rep((1, H)),
        ],
        out_specs=[
            pl.BlockSpec((_BB, H), lambda i: (i, 0)),
            pl.BlockSpec((_BB, N, N), lambda i: (i, 0, 0)),
        ],
        out_shape=[
            jax.ShapeDtypeStruct((B, H), jnp.float32),
            jax.ShapeDtypeStruct((B, N, N), jnp.float32),
        ],
        compiler_params=pltpu.CompilerParams(
            dimension_semantics=("parallel",)),
    )(bf(x), bf(W_emb), b_emb.reshape(1, H),
      bf(W_ap), b_msg.reshape(_ITERS, 1, H),
      bf(W_zr), b_zr.reshape(_ITERS, 1, 2 * H),
      bf(W_cand), bh.reshape(_ITERS, 1, H),
      bf(W_ro), b_ro.reshape(1, H))
    return (out, A)


# speedup vs baseline: 2.1877x; 1.0432x over previous
import jax
import jax.numpy as jnp
from jax.experimental import pallas as pl
from jax.experimental.pallas import tpu as pltpu

_HIDDEN = 64
_N = 128
_ITERS = 2
_BB = 32
_LOG2E = 1.4426950408889634


def _dot(a, b):
    return jax.lax.dot_general(a, b, (((1,), (0,)), ((), ())),
                               preferred_element_type=jnp.float32)


def _bf(v):
    return v.astype(jnp.bfloat16)


def _mpnn_kernel(x_ref, Wemb_ref, bemb_ref, Wap_ref, bmsg_ref,
                 Wzr_ref, bzr_ref, Wcand_ref, bh_ref, Wro_ref, bro_ref,
                 out_ref, A_ref):
    H = _HIDDEN
    h = jnp.tanh(_dot(x_ref[...], Wemb_ref[...]) + bemb_ref[...])
    ones_blk = jnp.ones((_N, H), jnp.bfloat16)
    for t in range(_ITERS):
        h16 = _bf(h)
        hp = _dot(h16, Wap_ref[t])
        hw16 = _bf(hp[:, :H])
        pre16 = _bf(hp[:, H:] + bmsg_ref[t])
        es = []
        for b in range(_BB):
            sl = slice(b * _N, (b + 1) * _N)
            logits = jax.lax.dot_general(
                hw16[sl, :], h16[sl, :], (((1,), (1,)), ((), ())),
                preferred_element_type=jnp.float32)
            es.append(jnp.exp2(logits))
        ss = []
        for b in range(_BB):
            sl = slice(b * _N, (b + 1) * _N)
            pre_aug = jnp.concatenate([pre16[sl, :], ones_blk], axis=1)
            ss.append(_dot(_bf(es[b]), pre_aug))
        msgs = []
        for b in range(_BB):
            inv = 1.0 / ss[b][:, H:]
            msgs.append(ss[b][:, :H] * inv)
            if t == _ITERS - 1:
                A_ref[b] = es[b] * jnp.concatenate([inv, inv], axis=1)
        msg = jnp.tanh(jnp.concatenate(msgs, axis=0))
        msg16 = _bf(msg)
        mh16 = jnp.concatenate([msg16, h16], axis=1)
        zr = jnp.tanh(_dot(mh16, Wzr_ref[t]) + bzr_ref[t])
        z = 0.5 + 0.5 * zr[:, :H]
        r = 0.5 + 0.5 * zr[:, H:]
        mrh16 = jnp.concatenate([msg16, _bf(r * h)], axis=1)
        htil = jnp.tanh(_dot(mrh16, Wcand_ref[t]) + bh_ref[t])
        h = h + z * (htil - h)
    pooled = jnp.concatenate(
        [jnp.sum(h[b * _N:(b + 1) * _N, :], axis=0, keepdims=True)
         for b in range(_BB)], axis=0)
    out_ref[...] = jnp.tanh(_dot(pooled, Wro_ref[...]) + bro_ref[...])


def kernel(jets, W_emb, b_emb, W_adj, W_msg, b_msg,
           Wz, Uz, bz, Wr, Ur, br, Wh, Uh, bh, W_ro, b_ro):
    B, N, F = jets.shape
    H = _HIDDEN
    x = jnp.concatenate([jets, jnp.ones((B, N, 1), jets.dtype)], axis=-1)
    x = x.reshape(B * N, F + 1)
    W_adj_s = W_adj * jnp.float32(_LOG2E / (float(H) ** 0.5))
    W_ap = jnp.concatenate([W_adj_s, W_msg], axis=2)
    W_zr = 0.5 * jnp.concatenate(
        [jnp.concatenate([Wz, Wr], axis=2),
         jnp.concatenate([Uz, Ur], axis=2)], axis=1)
    b_zr = 0.5 * jnp.concatenate([bz, br], axis=1)
    W_cand = jnp.concatenate([Wh, Uh], axis=1)
    bf = lambda v: v.astype(jnp.bfloat16)

    def rep(ix):
        return pl.BlockSpec(ix, lambda i: (0,) * len(ix))

    out, A = pl.pallas_call(
        _mpnn_kernel,
        grid=(B // _BB,),
        in_specs=[
            pl.BlockSpec((_BB * N, F + 1), lambda i: (i, 0)),
            rep((F + 1, H)),
            rep((1, H)),
            rep((_ITERS, H, 2 * H)),
            rep((_ITERS, 1, H)),
            rep((_ITERS, 2 * H, 2 * H)),
            rep((_ITERS, 1, 2 * H)),
            rep((_ITERS, 2 * H, H)),
            rep((_ITERS, 1, H)),
            rep((H, H)),
            rep((1, H)),
        ],
        out_specs=[
            pl.BlockSpec((_BB, H), lambda i: (i, 0)),
            pl.BlockSpec((_BB, N, N), lambda i: (i, 0, 0)),
        ],
        out_shape=[
            jax.ShapeDtypeStruct((B, H), jnp.float32),
            jax.ShapeDtypeStruct((B, N, N), jnp.float32),
        ],
        compiler_params=pltpu.CompilerParams(
            dimension_semantics=("parallel",)),
    )(bf(x), bf(W_emb), b_emb.reshape(1, H),
      bf(W_ap), b_msg.reshape(_ITERS, 1, H),
      bf(W_zr), b_zr.reshape(_ITERS, 1, 2 * H),
      bf(W_cand), bh.reshape(_ITERS, 1, H),
      bf(W_ro), b_ro.reshape(1, H))
    return (out, A)


# speedup vs baseline: 2.2074x; 1.0090x over previous
import jax
import jax.numpy as jnp
from jax.experimental import pallas as pl
from jax.experimental.pallas import tpu as pltpu

_HIDDEN = 64
_N = 128
_ITERS = 2
_BB = 64
_LOG2E = 1.4426950408889634


def _dot(a, b):
    return jax.lax.dot_general(a, b, (((1,), (0,)), ((), ())),
                               preferred_element_type=jnp.float32)


def _bf(v):
    return v.astype(jnp.bfloat16)


def _mpnn_kernel(x_ref, Wemb_ref, bemb_ref, Wap_ref, bmsg_ref,
                 Wzr_ref, bzr_ref, Wcand_ref, bh_ref, Wro_ref, bro_ref,
                 out_ref, A_ref):
    H = _HIDDEN
    h = jnp.tanh(_dot(x_ref[...], Wemb_ref[...]) + bemb_ref[...])
    ones_blk = jnp.ones((_N, H), jnp.bfloat16)
    for t in range(_ITERS):
        h16 = _bf(h)
        hp = _dot(h16, Wap_ref[t])
        hw16 = _bf(hp[:, :H])
        pre16 = _bf(hp[:, H:] + bmsg_ref[t])
        es = []
        for b in range(_BB):
            sl = slice(b * _N, (b + 1) * _N)
            logits = jax.lax.dot_general(
                hw16[sl, :], h16[sl, :], (((1,), (1,)), ((), ())),
                preferred_element_type=jnp.float32)
            es.append(jnp.exp2(logits))
        ss = []
        for b in range(_BB):
            sl = slice(b * _N, (b + 1) * _N)
            pre_aug = jnp.concatenate([pre16[sl, :], ones_blk], axis=1)
            ss.append(_dot(_bf(es[b]), pre_aug))
        msgs = []
        for b in range(_BB):
            inv = 1.0 / ss[b][:, H:]
            msgs.append(ss[b][:, :H] * inv)
            if t == _ITERS - 1:
                A_ref[b] = es[b] * jnp.concatenate([inv, inv], axis=1)
        msg = jnp.tanh(jnp.concatenate(msgs, axis=0))
        msg16 = _bf(msg)
        mh16 = jnp.concatenate([msg16, h16], axis=1)
        zr = jnp.tanh(_dot(mh16, Wzr_ref[t]) + bzr_ref[t])
        z = 0.5 + 0.5 * zr[:, :H]
        r = 0.5 + 0.5 * zr[:, H:]
        mrh16 = jnp.concatenate([msg16, _bf(r * h)], axis=1)
        htil = jnp.tanh(_dot(mrh16, Wcand_ref[t]) + bh_ref[t])
        h = h + z * (htil - h)
    pooled = jnp.concatenate(
        [jnp.sum(h[b * _N:(b + 1) * _N, :], axis=0, keepdims=True)
         for b in range(_BB)], axis=0)
    out_ref[...] = jnp.tanh(_dot(pooled, Wro_ref[...]) + bro_ref[...])


def kernel(jets, W_emb, b_emb, W_adj, W_msg, b_msg,
           Wz, Uz, bz, Wr, Ur, br, Wh, Uh, bh, W_ro, b_ro):
    B, N, F = jets.shape
    H = _HIDDEN
    x = jnp.concatenate([jets, jnp.ones((B, N, 1), jets.dtype)], axis=-1)
    x = x.reshape(B * N, F + 1)
    W_adj_s = W_adj * jnp.float32(_LOG2E / (float(H) ** 0.5))
    W_ap = jnp.concatenate([W_adj_s, W_msg], axis=2)
    W_zr = 0.5 * jnp.concatenate(
        [jnp.concatenate([Wz, Wr], axis=2),
         jnp.concatenate([Uz, Ur], axis=2)], axis=1)
    b_zr = 0.5 * jnp.concatenate([bz, br], axis=1)
    W_cand = jnp.concatenate([Wh, Uh], axis=1)
    bf = lambda v: v.astype(jnp.bfloat16)

    def rep(ix):
        return pl.BlockSpec(ix, lambda i: (0,) * len(ix))

    out, A = pl.pallas_call(
        _mpnn_kernel,
        grid=(B // _BB,),
        in_specs=[
            pl.BlockSpec((_BB * N, F + 1), lambda i: (i, 0)),
            rep((F + 1, H)),
            rep((1, H)),
            rep((_ITERS, H, 2 * H)),
            rep((_ITERS, 1, H)),
            rep((_ITERS, 2 * H, 2 * H)),
            rep((_ITERS, 1, 2 * H)),
            rep((_ITERS, 2 * H, H)),
            rep((_ITERS, 1, H)),
            rep((H, H)),
            rep((1, H)),
        ],
        out_specs=[
            pl.BlockSpec((_BB, H), lambda i: (i, 0)),
            pl.BlockSpec((_BB, N, N), lambda i: (i, 0, 0)),
        ],
        out_shape=[
            jax.ShapeDtypeStruct((B, H), jnp.float32),
            jax.ShapeDtypeStruct((B, N, N), jnp.float32),
        ],
        compiler_params=pltpu.CompilerParams(
            dimension_semantics=("parallel",)),
    )(bf(x), bf(W_emb), b_emb.reshape(1, H),
      bf(W_ap), b_msg.reshape(_ITERS, 1, H),
      bf(W_zr), b_zr.reshape(_ITERS, 1, 2 * H),
      bf(W_cand), bh.reshape(_ITERS, 1, H),
      bf(W_ro), b_ro.reshape(1, H))
    return (out, A)


# speedup vs baseline: 2.5576x; 1.1586x over previous
import jax
import jax.numpy as jnp
from jax.experimental import pallas as pl
from jax.experimental.pallas import tpu as pltpu

_HIDDEN = 64
_N = 128
_ITERS = 2
_BB = 64
_LOG2E = 1.4426950408889634


def _dot(a, b):
    return jax.lax.dot_general(a, b, (((1,), (0,)), ((), ())),
                               preferred_element_type=jnp.float32)


def _bf(v):
    return v.astype(jnp.bfloat16)


def _mpnn_kernel(x_ref, Wemb_ref, bemb_ref, Wadj_ref, Wmsg_ref, bmsg_ref,
                 Wz_ref, Uz_ref, bz_ref, Wr_ref, Ur_ref, br_ref,
                 Wh_ref, Uh_ref, bh_ref, Wro_ref, bro_ref,
                 out_ref, A_ref):
    H = _HIDDEN
    h = jnp.tanh(_dot(x_ref[...], _bf(Wemb_ref[...])) + bemb_ref[...])
    ones_blk = jnp.ones((_N, H), jnp.bfloat16)
    for t in range(_ITERS):
        Wap16 = _bf(jnp.concatenate(
            [Wadj_ref[t] * jnp.float32(_LOG2E / (float(H) ** 0.5)),
             Wmsg_ref[t]], axis=1))
        Wzr16 = _bf(0.5 * jnp.concatenate(
            [jnp.concatenate([Wz_ref[t], Wr_ref[t]], axis=1),
             jnp.concatenate([Uz_ref[t], Ur_ref[t]], axis=1)], axis=0))
        bzr = 0.5 * jnp.concatenate([bz_ref[t], br_ref[t]], axis=1)
        Wcand16 = _bf(jnp.concatenate([Wh_ref[t], Uh_ref[t]], axis=0))

        h16 = _bf(h)
        hp = _dot(h16, Wap16)
        hw16 = _bf(hp[:, :H])
        pre16 = _bf(hp[:, H:] + bmsg_ref[t])
        es = []
        for b in range(_BB):
            sl = slice(b * _N, (b + 1) * _N)
            logits = jax.lax.dot_general(
                hw16[sl, :], h16[sl, :], (((1,), (1,)), ((), ())),
                preferred_element_type=jnp.float32)
            es.append(jnp.exp2(logits))
        ss = []
        for b in range(_BB):
            sl = slice(b * _N, (b + 1) * _N)
            pre_aug = jnp.concatenate([pre16[sl, :], ones_blk], axis=1)
            ss.append(_dot(_bf(es[b]), pre_aug))
        msgs = []
        for b in range(_BB):
            inv = 1.0 / ss[b][:, H:]
            msgs.append(ss[b][:, :H] * inv)
            if t == _ITERS - 1:
                A_ref[b] = es[b] * jnp.concatenate([inv, inv], axis=1)
        msg = jnp.tanh(jnp.concatenate(msgs, axis=0))
        msg16 = _bf(msg)
        mh16 = jnp.concatenate([msg16, h16], axis=1)
        zr = jnp.tanh(_dot(mh16, Wzr16) + bzr)
        z = 0.5 + 0.5 * zr[:, :H]
        r = 0.5 + 0.5 * zr[:, H:]
        mrh16 = jnp.concatenate([msg16, _bf(r * h)], axis=1)
        htil = jnp.tanh(_dot(mrh16, Wcand16) + bh_ref[t])
        h = h + z * (htil - h)
    pooled = jnp.concatenate(
        [jnp.sum(h[b * _N:(b + 1) * _N, :], axis=0, keepdims=True)
         for b in range(_BB)], axis=0)
    out_ref[...] = jnp.tanh(_dot(_bf(pooled), _bf(Wro_ref[...])) + bro_ref[...])


def kernel(jets, W_emb, b_emb, W_adj, W_msg, b_msg,
           Wz, Uz, bz, Wr, Ur, br, Wh, Uh, bh, W_ro, b_ro):
    B, N, F = jets.shape
    H = _HIDDEN
    x = jnp.concatenate([jets, jnp.ones((B, N, 1), jets.dtype)], axis=-1)
    x = x.reshape(B * N, F + 1).astype(jnp.bfloat16)

    def rep(ix):
        return pl.BlockSpec(ix, lambda i: (0,) * len(ix))

    out, A = pl.pallas_call(
        _mpnn_kernel,
        grid=(B // _BB,),
        in_specs=[
            pl.BlockSpec((_BB * N, F + 1), lambda i: (i, 0)),
            rep((F + 1, H)),
            rep((1, H)),
            rep((_ITERS, H, H)),
            rep((_ITERS, H, H)),
            rep((_ITERS, 1, H)),
            rep((_ITERS, H, H)), rep((_ITERS, H, H)), rep((_ITERS, 1, H)),
            rep((_ITERS, H, H)), rep((_ITERS, H, H)), rep((_ITERS, 1, H)),
            rep((_ITERS, H, H)), rep((_ITERS, H, H)), rep((_ITERS, 1, H)),
            rep((H, H)),
            rep((1, H)),
        ],
        out_specs=[
            pl.BlockSpec((_BB, H), lambda i: (i, 0)),
            pl.BlockSpec((_BB, N, N), lambda i: (i, 0, 0)),
        ],
        out_shape=[
            jax.ShapeDtypeStruct((B, H), jnp.float32),
            jax.ShapeDtypeStruct((B, N, N), jnp.float32),
        ],
        compiler_params=pltpu.CompilerParams(
            dimension_semantics=("parallel",)),
    )(x, W_emb, b_emb.reshape(1, H),
      W_adj, W_msg, b_msg.reshape(_ITERS, 1, H),
      Wz, Uz, bz.reshape(_ITERS, 1, H),
      Wr, Ur, br.reshape(_ITERS, 1, H),
      Wh, Uh, bh.reshape(_ITERS, 1, H),
      W_ro, b_ro.reshape(1, H))
    return (out, A)


# speedup vs baseline: 2.8796x; 1.1259x over previous
import jax
import jax.numpy as jnp
from jax.experimental import pallas as pl
from jax.experimental.pallas import tpu as pltpu

_HIDDEN = 64
_N = 128
_ITERS = 2
_BB = 64
_LOG2E = 1.4426950408889634
_NFEAT = 7


def _dot(a, b):
    return jax.lax.dot_general(a, b, (((1,), (0,)), ((), ())),
                               preferred_element_type=jnp.float32)


def _bf(v):
    return v.astype(jnp.bfloat16)


def _mpnn_kernel(x_ref, Wemb_ref, bemb_ref, Wadj_ref, Wmsg_ref, bmsg_ref,
                 Wz_ref, Uz_ref, bz_ref, Wr_ref, Ur_ref, br_ref,
                 Wh_ref, Uh_ref, bh_ref, Wro_ref, bro_ref,
                 out_ref, A_ref):
    H = _HIDDEN
    beff = bemb_ref[...] + Wemb_ref[_NFEAT:, :]
    h = jnp.tanh(_dot(_bf(x_ref[...]), _bf(Wemb_ref[: _NFEAT, :])) + beff)
    ones_blk = jnp.ones((_N, H), jnp.bfloat16)
    for t in range(_ITERS):
        Wap16 = _bf(jnp.concatenate(
            [Wadj_ref[t] * jnp.float32(_LOG2E / (float(H) ** 0.5)),
             Wmsg_ref[t]], axis=1))
        Wzr16 = _bf(0.5 * jnp.concatenate(
            [jnp.concatenate([Wz_ref[t], Wr_ref[t]], axis=1),
             jnp.concatenate([Uz_ref[t], Ur_ref[t]], axis=1)], axis=0))
        bzr = 0.5 * jnp.concatenate([bz_ref[t], br_ref[t]], axis=1)
        Wcand16 = _bf(jnp.concatenate([Wh_ref[t], Uh_ref[t]], axis=0))

        h16 = _bf(h)
        hp = _dot(h16, Wap16)
        hw16 = _bf(hp[:, :H])
        pre16 = _bf(hp[:, H:] + bmsg_ref[t])
        es = []
        for b in range(_BB):
            sl = slice(b * _N, (b + 1) * _N)
            logits = jax.lax.dot_general(
                hw16[sl, :], h16[sl, :], (((1,), (1,)), ((), ())),
                preferred_element_type=jnp.float32)
            es.append(jnp.exp2(logits))
        ss = []
        for b in range(_BB):
            sl = slice(b * _N, (b + 1) * _N)
            pre_aug = jnp.concatenate([pre16[sl, :], ones_blk], axis=1)
            ss.append(_dot(_bf(es[b]), pre_aug))
        msgs = []
        for b in range(_BB):
            inv = 1.0 / ss[b][:, H:]
            msgs.append(ss[b][:, :H] * inv)
            if t == _ITERS - 1:
                A_ref[b] = es[b] * jnp.concatenate([inv, inv], axis=1)
        msg = jnp.tanh(jnp.concatenate(msgs, axis=0))
        msg16 = _bf(msg)
        mh16 = jnp.concatenate([msg16, h16], axis=1)
        zr = jnp.tanh(_dot(mh16, Wzr16) + bzr)
        z = 0.5 + 0.5 * zr[:, :H]
        r = 0.5 + 0.5 * zr[:, H:]
        mrh16 = jnp.concatenate([msg16, _bf(r * h)], axis=1)
        htil = jnp.tanh(_dot(mrh16, Wcand16) + bh_ref[t])
        h = h + z * (htil - h)
    pooled = jnp.concatenate(
        [jnp.sum(h[b * _N:(b + 1) * _N, :], axis=0, keepdims=True)
         for b in range(_BB)], axis=0)
    out_ref[...] = jnp.tanh(_dot(_bf(pooled), _bf(Wro_ref[...])) + bro_ref[...])


def kernel(jets, W_emb, b_emb, W_adj, W_msg, b_msg,
           Wz, Uz, bz, Wr, Ur, br, Wh, Uh, bh, W_ro, b_ro):
    B, N, F = jets.shape
    H = _HIDDEN
    x = jets.reshape(B * N, F)

    def rep(ix):
        return pl.BlockSpec(ix, lambda i: (0,) * len(ix))

    out, A = pl.pallas_call(
        _mpnn_kernel,
        grid=(B // _BB,),
        in_specs=[
            pl.BlockSpec((_BB * N, F), lambda i: (i, 0)),
            rep((F + 1, H)),
            rep((1, H)),
            rep((_ITERS, H, H)),
            rep((_ITERS, H, H)),
            rep((_ITERS, 1, H)),
            rep((_ITERS, H, H)), rep((_ITERS, H, H)), rep((_ITERS, 1, H)),
            rep((_ITERS, H, H)), rep((_ITERS, H, H)), rep((_ITERS, 1, H)),
            rep((_ITERS, H, H)), rep((_ITERS, H, H)), rep((_ITERS, 1, H)),
            rep((H, H)),
            rep((1, H)),
        ],
        out_specs=[
            pl.BlockSpec((_BB, H), lambda i: (i, 0)),
            pl.BlockSpec((_BB, N, N), lambda i: (i, 0, 0)),
        ],
        out_shape=[
            jax.ShapeDtypeStruct((B, H), jnp.float32),
            jax.ShapeDtypeStruct((B, N, N), jnp.float32),
        ],
        compiler_params=pltpu.CompilerParams(
            dimension_semantics=("parallel",)),
    )(x, W_emb, b_emb.reshape(1, H),
      W_adj, W_msg, b_msg.reshape(_ITERS, 1, H),
      Wz, Uz, bz.reshape(_ITERS, 1, H),
      Wr, Ur, br.reshape(_ITERS, 1, H),
      Wh, Uh, bh.reshape(_ITERS, 1, H),
      W_ro, b_ro.reshape(1, H))
    return (out, A)
